# Initial kernel scaffold; baseline (speedup 1.0000x reference)
import jax
import jax.numpy as jnp
from jax.experimental import pallas as pl


def kernel(feats, edge_index, W_conv, b_conv, gate_W, gate_b, W1, b1, W2, b2, bn_gamma, bn_beta):
    raise NotImplementedError("write your pallas kernel here")


# speedup vs baseline: 2.9125x; 2.9125x over previous
import functools

import jax
import jax.numpy as jnp
from jax import lax
from jax.experimental import pallas as pl
from jax.experimental.pallas import tpu as pltpu
from jax.experimental.pallas import tpu_sc as plsc

N = 10000
E = 320000
D = 128
H = 256
NUM_EXPERTS = 64
TOP_K = 2

NC = 2
NS = 16
LANES = 16
NW = NC * NS

EDGES_PER_TILE = E // NW
EDGE_BATCH = 80
NPAD = 10240
ROWS_PER_TILE = NPAD // NS
ZROWS = 128


def _mesh():
  return plsc.VectorSubcoreMesh(
      core_axis_name="c", subcore_axis_name="s", num_cores=NC,
      num_subcores=NS)


def _degrees_body(src_hbm, dst_hbm, out, sidx, didx, ones_v, zbuf,
                  cnt_out_sh, cnt_in_sh):
  c = lax.axis_index("c")
  s = lax.axis_index("s")

  zeros16 = jnp.zeros((LANES,), jnp.float32)

  def zero_body(i, _):
    zbuf[pl.ds(i * LANES, LANES)] = zeros16
    return 0

  lax.fori_loop(0, N // LANES, zero_body, 0)

  @pl.when(s == 0)
  def _():
    pltpu.sync_copy(zbuf, cnt_out_sh)

  @pl.when(s == 1)
  def _():
    pltpu.sync_copy(zbuf, cnt_in_sh)

  def ones_body(i, _):
    ones_v[pl.ds(i * LANES, LANES)] = jnp.ones((LANES,), jnp.float32)
    return 0

  lax.fori_loop(0, EDGE_BATCH // LANES, ones_body, 0)
  plsc.subcore_barrier()

  def count_body(i, _):
    base = c * (E // NC) + s * EDGES_PER_TILE + i * EDGE_BATCH
    pltpu.sync_copy(src_hbm.at[pl.ds(base, EDGE_BATCH)], sidx)
    pltpu.sync_copy(dst_hbm.at[pl.ds(base, EDGE_BATCH)], didx)
    pltpu.sync_copy(ones_v, cnt_out_sh.at[sidx], add=True)
    pltpu.sync_copy(ones_v, cnt_in_sh.at[didx], add=True)
    return 0

  lax.fori_loop(0, EDGES_PER_TILE // EDGE_BATCH, count_body, 0)
  plsc.subcore_barrier()

  @pl.when(s == 0)
  def _():
    pltpu.sync_copy(cnt_out_sh, out.at[c, 0])

  @pl.when(s == 1)
  def _():
    pltpu.sync_copy(cnt_in_sh, out.at[c, 1])


def _degrees(src, dst):
  k = pl.kernel(
      _degrees_body,
      out_type=jax.ShapeDtypeStruct((NC, 2, N), jnp.float32),
      mesh=_mesh(),
      scratch_types=[
          pltpu.VMEM((EDGE_BATCH,), jnp.int32),
          pltpu.VMEM((EDGE_BATCH,), jnp.int32),
          pltpu.VMEM((EDGE_BATCH,), jnp.float32),
          pltpu.VMEM((N,), jnp.float32),
          pltpu.MemorySpace.VMEM_SHARED((N,), jnp.float32),
          pltpu.MemorySpace.VMEM_SHARED((N,), jnp.float32),
      ],
  )
  return k(src, dst)


def _scale_body(cnts_ref, feats_ref, hpre_ref, isr_in_ref):
  cnts = jnp.sum(cnts_ref[...], axis=0)
  deg = jnp.maximum(cnts, 1.0)
  isr = lax.rsqrt(deg)
  hpre_ref[...] = feats_ref[...] * isr[0][:, None]
  isr_in_ref[...] = isr[1][:, None]


def _scale(cnts, feats):
  return pl.pallas_call(
      _scale_body,
      out_shape=(
          jax.ShapeDtypeStruct((N, D), jnp.float32),
          jax.ShapeDtypeStruct((N, 1), jnp.float32),
      ),
  )(cnts, feats)


def _agg_body(hpre, src_hbm, dst_hbm, out, sidx, didx, rows, zbuf, shared,
              sem):
  c = lax.axis_index("c")
  s = lax.axis_index("s")

  zeros16 = jnp.zeros((LANES,), jnp.float32)

  def zero_body(i, _):
    def zcol(j, _):
      zbuf[i, pl.ds(j * LANES, LANES)] = zeros16
      return 0
    lax.fori_loop(0, D // LANES, zcol, 0)
    return 0

  lax.fori_loop(0, ZROWS, zero_body, 0)
  for j in range(ROWS_PER_TILE // ZROWS):
    pltpu.sync_copy(zbuf, shared.at[pl.ds(s * ROWS_PER_TILE + j * ZROWS, ZROWS)])
  plsc.subcore_barrier()

  def edge_body(i, _):
    base = c * (E // NC) + s * EDGES_PER_TILE + i * EDGE_BATCH
    pltpu.sync_copy(src_hbm.at[pl.ds(base, EDGE_BATCH)], sidx)
    pltpu.sync_copy(dst_hbm.at[pl.ds(base, EDGE_BATCH)], didx)
    pltpu.async_copy(hpre.at[sidx], rows, sem).wait()
    pltpu.sync_copy(rows, shared.at[didx], add=True)
    return 0

  lax.fori_loop(0, EDGES_PER_TILE // EDGE_BATCH, edge_body, 0)
  plsc.subcore_barrier()

  for j in range(ROWS_PER_TILE // ZROWS):
    r0 = s * ROWS_PER_TILE + j * ZROWS
    pltpu.sync_copy(shared.at[pl.ds(r0, ZROWS)], out.at[c, pl.ds(r0, ZROWS)])


def _aggregate(hpre, src, dst):
  k = pl.kernel(
      _agg_body,
      out_type=jax.ShapeDtypeStruct((NC, NPAD, D), jnp.float32),
      mesh=_mesh(),
      scratch_types=[
          pltpu.VMEM((EDGE_BATCH,), jnp.int32),
          pltpu.VMEM((EDGE_BATCH,), jnp.int32),
          pltpu.VMEM((EDGE_BATCH, D), jnp.float32),
          pltpu.VMEM((ZROWS, D), jnp.float32),
          pltpu.MemorySpace.VMEM_SHARED((NPAD, D), jnp.float32),
          pltpu.SemaphoreType.DMA,
      ],
  )
  return k(hpre, src, dst)


def _conv_gate_body(p_ref, isr_ref, wc_ref, bc_ref, gw_ref, gb_ref,
                    h_ref, idx_ref, g_ref):
  agg = (p_ref[0] + p_ref[1]) * isr_ref[...]
  h = jnp.dot(agg, wc_ref[...], preferred_element_type=jnp.float32)
  h = h + bc_ref[...][None, :]
  h_ref[...] = h
  logits = jnp.dot(h, gw_ref[...], preferred_element_type=jnp.float32)
  logits = logits + gb_ref[...][None, :]
  iota = lax.broadcasted_iota(jnp.int32, (N, NUM_EXPERTS), 1)
  m1 = jnp.max(logits, axis=1, keepdims=True)
  i1 = jnp.min(jnp.where(logits == m1, iota, NUM_EXPERTS), axis=1,
               keepdims=True)
  masked = jnp.where(iota == i1, -jnp.inf, logits)
  m2 = jnp.max(masked, axis=1, keepdims=True)
  i2 = jnp.min(jnp.where(masked == m2, iota, NUM_EXPERTS), axis=1,
               keepdims=True)
  e2 = jnp.exp(m2 - m1)
  denom = 1.0 + e2
  g1 = 1.0 / denom
  g2 = e2 / denom
  idx_ref[...] = jnp.concatenate([i1, i2], axis=1)
  g_ref[...] = jnp.concatenate([g1, g2], axis=1)


def _conv_gate(parts, isr_in, W_conv, b_conv, gate_W, gate_b):
  return pl.pallas_call(
      _conv_gate_body,
      out_shape=(
          jax.ShapeDtypeStruct((N, D), jnp.float32),
          jax.ShapeDtypeStruct((N, TOP_K), jnp.int32),
          jax.ShapeDtypeStruct((N, TOP_K), jnp.float32),
      ),
  )(parts, isr_in, W_conv, b_conv, gate_W, gate_b)


A2 = N * TOP_K
BB = 256
PB = 36864
NB = PB // BB
GB = 128
GROWS = PB // NW
NPAD2 = 10240
CTOK = NPAD2 // NW
CB = 40


def _gather_rows_body(h_hbm, tok_hbm, out, tidx, rows, sem):
  c = lax.axis_index("c")
  s = lax.axis_index("s")
  w = s * NC + c
  base = w * GROWS

  def body(i, _):
    r0 = base + i * GB
    pltpu.sync_copy(tok_hbm.at[pl.ds(r0, GB)], tidx)
    pltpu.async_copy(h_hbm.at[tidx], rows, sem).wait()
    pltpu.sync_copy(rows, out.at[pl.ds(r0, GB)])
    return 0

  lax.fori_loop(0, GROWS // GB, body, 0)


def _gather_rows(h, tok_padded):
  k = pl.kernel(
      _gather_rows_body,
      out_type=jax.ShapeDtypeStruct((PB, D), jnp.float32),
      mesh=_mesh(),
      scratch_types=[
          pltpu.VMEM((GB,), jnp.int32),
          pltpu.VMEM((GB, D), jnp.float32),
          pltpu.SemaphoreType.DMA,
      ],
  )
  return k(h, tok_padded)


def _ffn_body(be_ref, x_ref, g_ref, w1_ref, b1_ref, w2_ref, b2_ref, y_ref):
  x = x_ref[...]
  a = jnp.dot(x, w1_ref[0], preferred_element_type=jnp.float32)
  a = a + b1_ref[0]
  a = 0.5 * a * (1.0 + lax.erf(a * 0.7071067811865476))
  y = jnp.dot(a, w2_ref[0], preferred_element_type=jnp.float32)
  y = y + b2_ref[0]
  y_ref[...] = y * g_ref[...]


def _ffn(x_sorted, gate_padded, block_expert, W1, b1, W2, b2):
  grid_spec = pltpu.PrefetchScalarGridSpec(
      num_scalar_prefetch=1,
      grid=(NB,),
      in_specs=[
          pl.BlockSpec((BB, D), lambda b, be: (b, 0)),
          pl.BlockSpec((BB, 1), lambda b, be: (b, 0)),
          pl.BlockSpec((1, D, H), lambda b, be: (be[b], 0, 0)),
          pl.BlockSpec((1, 1, H), lambda b, be: (be[b], 0, 0)),
          pl.BlockSpec((1, H, D), lambda b, be: (be[b], 0, 0)),
          pl.BlockSpec((1, 1, D), lambda b, be: (be[b], 0, 0)),
      ],
      out_specs=pl.BlockSpec((BB, D), lambda b, be: (b, 0)),
  )
  return pl.pallas_call(
      _ffn_body,
      grid_spec=grid_spec,
      out_shape=jax.ShapeDtypeStruct((PB, D), jnp.float32),
  )(block_expert, x_sorted, gate_padded.reshape(PB, 1),
    W1, b1.reshape(NUM_EXPERTS, 1, H), W2, b2.reshape(NUM_EXPERTS, 1, D))


def _combine_body(y_hbm, inv_hbm, out, iidx, rows, obuf, sem):
  c = lax.axis_index("c")
  s = lax.axis_index("s")
  w = s * NC + c
  tbase = w * CTOK

  def body(j, _):
    pltpu.sync_copy(inv_hbm.at[pl.ds(2 * tbase + j * 2 * CB, 2 * CB)], iidx)
    pltpu.async_copy(y_hbm.at[iidx], rows, sem).wait()

    def tok(i, _):
      def col(cc, _):
        a = rows[2 * i, pl.ds(cc * LANES, LANES)]
        b = rows[2 * i + 1, pl.ds(cc * LANES, LANES)]
        obuf[i, pl.ds(cc * LANES, LANES)] = a + b
        return 0
      lax.fori_loop(0, D // LANES, col, 0)
      return 0

    lax.fori_loop(0, CB, tok, 0)
    pltpu.sync_copy(obuf, out.at[pl.ds(tbase + j * CB, CB)])
    return 0

  lax.fori_loop(0, CTOK // CB, body, 0)


def _combine(y_sorted, invpos_padded):
  k = pl.kernel(
      _combine_body,
      out_type=jax.ShapeDtypeStruct((NPAD2, D), jnp.float32),
      mesh=_mesh(),
      scratch_types=[
          pltpu.VMEM((2 * CB,), jnp.int32),
          pltpu.VMEM((2 * CB, D), jnp.float32),
          pltpu.VMEM((CB, D), jnp.float32),
          pltpu.SemaphoreType.DMA,
      ],
  )
  return k(y_sorted, invpos_padded)


def _bn_body(h_ref, moe_ref, gamma_ref, beta_ref, out_ref):
  z = h_ref[...] + moe_ref[...]
  mean = jnp.mean(z, axis=0, keepdims=True)
  zc = z - mean
  var = jnp.mean(zc * zc, axis=0, keepdims=True)
  out_ref[...] = zc * lax.rsqrt(var + 1e-5) * gamma_ref[...][None, :] \
      + beta_ref[...][None, :]


def _batchnorm(h, moe, gamma, beta):
  return pl.pallas_call(
      _bn_body,
      out_shape=jax.ShapeDtypeStruct((N, D), jnp.float32),
  )(h, moe, gamma, beta)


def kernel(feats, edge_index, W_conv, b_conv, gate_W, gate_b, W1, b1, W2, b2,
           bn_gamma, bn_beta):
  src = edge_index[0]
  dst = edge_index[1]
  cnts = _degrees(src, dst)
  hpre, isr_in = _scale(cnts, feats)
  parts = _aggregate(hpre, src, dst)[:, :N, :]
  h, idx2, gates = _conv_gate(parts, isr_in, W_conv, b_conv, gate_W, gate_b)

  ex = idx2.reshape(A2)
  gflat = gates.reshape(A2)
  perm = jnp.argsort(ex, stable=True).astype(jnp.int32)
  ex_sorted = jnp.take(ex, perm)
  offsets = jnp.searchsorted(ex_sorted, jnp.arange(NUM_EXPERTS, dtype=jnp.int32),
                             side="left").astype(jnp.int32)
  counts = jnp.diff(jnp.append(offsets, A2))
  pcounts = ((counts + BB - 1) // BB) * BB
  poffsets = jnp.concatenate([jnp.zeros((1,), jnp.int32),
                              jnp.cumsum(pcounts)]).astype(jnp.int32)
  shift = poffsets[:NUM_EXPERTS] - offsets
  pos = jnp.arange(A2, dtype=jnp.int32) + jnp.take(shift, ex_sorted)
  tok_padded = jnp.zeros((PB,), jnp.int32).at[pos].set(perm // TOP_K)
  gate_padded = jnp.zeros((PB,), jnp.float32).at[pos].set(jnp.take(gflat, perm))
  invpos = jnp.zeros((A2,), jnp.int32).at[perm].set(pos)
  invpos_padded = jnp.concatenate(
      [invpos, jnp.zeros((TOP_K * NPAD2 - A2,), jnp.int32)])
  block_expert = jnp.clip(
      jnp.searchsorted(poffsets, jnp.arange(NB, dtype=jnp.int32) * BB,
                       side="right").astype(jnp.int32) - 1, 0, NUM_EXPERTS - 1)

  x_sorted = _gather_rows(h, tok_padded)
  y_sorted = _ffn(x_sorted, gate_padded, block_expert, W1, b1, W2, b2)
  moe = _combine(y_sorted, invpos_padded)[:N]
  return _batchnorm(h, moe, bn_gamma, bn_beta)


# speedup vs baseline: 4.3098x; 1.4797x over previous
import functools

import jax
import jax.numpy as jnp
from jax import lax
from jax.experimental import pallas as pl
from jax.experimental.pallas import tpu as pltpu
from jax.experimental.pallas import tpu_sc as plsc

N = 10000
E = 320000
D = 128
H = 256
NUM_EXPERTS = 64
TOP_K = 2

NC = 2
NS = 16
LANES = 16
NW = NC * NS

EDGES_PER_TILE = E // NW
EDGE_BATCH = 80
NPAD = 10240
ROWS_PER_TILE = NPAD // NS
ZROWS = 128


def _mesh():
  return plsc.VectorSubcoreMesh(
      core_axis_name="c", subcore_axis_name="s", num_cores=NC,
      num_subcores=NS)


def _degrees_body(src_hbm, dst_hbm, out, sidx, didx, ones_v, zbuf,
                  cnt_out_sh, cnt_in_sh):
  c = lax.axis_index("c")
  s = lax.axis_index("s")

  zeros16 = jnp.zeros((LANES,), jnp.float32)

  def zero_body(i, _):
    zbuf[pl.ds(i * LANES, LANES)] = zeros16
    return 0

  lax.fori_loop(0, N // LANES, zero_body, 0)

  @pl.when(s == 0)
  def _():
    pltpu.sync_copy(zbuf, cnt_out_sh)

  @pl.when(s == 1)
  def _():
    pltpu.sync_copy(zbuf, cnt_in_sh)

  def ones_body(i, _):
    ones_v[pl.ds(i * LANES, LANES)] = jnp.ones((LANES,), jnp.float32)
    return 0

  lax.fori_loop(0, EDGE_BATCH // LANES, ones_body, 0)
  plsc.subcore_barrier()

  def count_body(i, _):
    base = c * (E // NC) + s * EDGES_PER_TILE + i * EDGE_BATCH
    pltpu.sync_copy(src_hbm.at[pl.ds(base, EDGE_BATCH)], sidx)
    pltpu.sync_copy(dst_hbm.at[pl.ds(base, EDGE_BATCH)], didx)
    pltpu.sync_copy(ones_v, cnt_out_sh.at[sidx], add=True)
    pltpu.sync_copy(ones_v, cnt_in_sh.at[didx], add=True)
    return 0

  lax.fori_loop(0, EDGES_PER_TILE // EDGE_BATCH, count_body, 0)
  plsc.subcore_barrier()

  @pl.when(s == 0)
  def _():
    pltpu.sync_copy(cnt_out_sh, out.at[c, 0])

  @pl.when(s == 1)
  def _():
    pltpu.sync_copy(cnt_in_sh, out.at[c, 1])


def _degrees(src, dst):
  k = pl.kernel(
      _degrees_body,
      out_type=jax.ShapeDtypeStruct((NC, 2, N), jnp.float32),
      mesh=_mesh(),
      scratch_types=[
          pltpu.VMEM((EDGE_BATCH,), jnp.int32),
          pltpu.VMEM((EDGE_BATCH,), jnp.int32),
          pltpu.VMEM((EDGE_BATCH,), jnp.float32),
          pltpu.VMEM((N,), jnp.float32),
          pltpu.MemorySpace.VMEM_SHARED((N,), jnp.float32),
          pltpu.MemorySpace.VMEM_SHARED((N,), jnp.float32),
      ],
  )
  return k(src, dst)


def _scale_body(cnts_ref, feats_ref, hpre_ref, isr_in_ref):
  cnts = jnp.sum(cnts_ref[...], axis=0)
  deg = jnp.maximum(cnts, 1.0)
  isr = lax.rsqrt(deg)
  hpre_ref[...] = feats_ref[...] * isr[0][:, None]
  isr_in_ref[...] = isr[1][:, None]


def _scale(cnts, feats):
  return pl.pallas_call(
      _scale_body,
      out_shape=(
          jax.ShapeDtypeStruct((N, D), jnp.float32),
          jax.ShapeDtypeStruct((N, 1), jnp.float32),
      ),
  )(cnts, feats)


def _agg_body(hpre, src_hbm, dst_hbm, out, sidx, didx, rows, zbuf, shared,
              sem):
  c = lax.axis_index("c")
  s = lax.axis_index("s")

  zeros16 = jnp.zeros((LANES,), jnp.float32)

  def zero_body(i, _):
    def zcol(j, _):
      zbuf[i, pl.ds(j * LANES, LANES)] = zeros16
      return 0
    lax.fori_loop(0, D // LANES, zcol, 0)
    return 0

  lax.fori_loop(0, ZROWS, zero_body, 0)
  for j in range(ROWS_PER_TILE // ZROWS):
    pltpu.sync_copy(zbuf, shared.at[pl.ds(s * ROWS_PER_TILE + j * ZROWS, ZROWS)])
  plsc.subcore_barrier()

  def edge_body(i, _):
    base = c * (E // NC) + s * EDGES_PER_TILE + i * EDGE_BATCH
    pltpu.sync_copy(src_hbm.at[pl.ds(base, EDGE_BATCH)], sidx)
    pltpu.sync_copy(dst_hbm.at[pl.ds(base, EDGE_BATCH)], didx)
    pltpu.async_copy(hpre.at[sidx], rows, sem).wait()
    pltpu.sync_copy(rows, shared.at[didx], add=True)
    return 0

  lax.fori_loop(0, EDGES_PER_TILE // EDGE_BATCH, edge_body, 0)
  plsc.subcore_barrier()

  for j in range(ROWS_PER_TILE // ZROWS):
    r0 = s * ROWS_PER_TILE + j * ZROWS
    pltpu.sync_copy(shared.at[pl.ds(r0, ZROWS)], out.at[c, pl.ds(r0, ZROWS)])


def _aggregate(hpre, src, dst):
  k = pl.kernel(
      _agg_body,
      out_type=jax.ShapeDtypeStruct((NC, NPAD, D), jnp.float32),
      mesh=_mesh(),
      scratch_types=[
          pltpu.VMEM((EDGE_BATCH,), jnp.int32),
          pltpu.VMEM((EDGE_BATCH,), jnp.int32),
          pltpu.VMEM((EDGE_BATCH, D), jnp.float32),
          pltpu.VMEM((ZROWS, D), jnp.float32),
          pltpu.MemorySpace.VMEM_SHARED((NPAD, D), jnp.float32),
          pltpu.SemaphoreType.DMA,
      ],
  )
  return k(hpre, src, dst)


def _conv_gate_body(p_ref, isr_ref, wc_ref, bc_ref, gw_ref, gb_ref,
                    h_ref, idx_ref, g_ref):
  agg = (p_ref[0] + p_ref[1]) * isr_ref[...]
  h = jnp.dot(agg, wc_ref[...], preferred_element_type=jnp.float32)
  h = h + bc_ref[...][None, :]
  h_ref[...] = h
  logits = jnp.dot(h, gw_ref[...], preferred_element_type=jnp.float32)
  logits = logits + gb_ref[...][None, :]
  iota = lax.broadcasted_iota(jnp.int32, (N, NUM_EXPERTS), 1)
  m1 = jnp.max(logits, axis=1, keepdims=True)
  i1 = jnp.min(jnp.where(logits == m1, iota, NUM_EXPERTS), axis=1,
               keepdims=True)
  masked = jnp.where(iota == i1, -jnp.inf, logits)
  m2 = jnp.max(masked, axis=1, keepdims=True)
  i2 = jnp.min(jnp.where(masked == m2, iota, NUM_EXPERTS), axis=1,
               keepdims=True)
  e2 = jnp.exp(m2 - m1)
  denom = 1.0 + e2
  g1 = 1.0 / denom
  g2 = e2 / denom
  idx_ref[...] = jnp.concatenate([i1, i2], axis=1)
  g_ref[...] = jnp.concatenate([g1, g2], axis=1)


def _conv_gate(parts, isr_in, W_conv, b_conv, gate_W, gate_b):
  return pl.pallas_call(
      _conv_gate_body,
      out_shape=(
          jax.ShapeDtypeStruct((N, D), jnp.float32),
          jax.ShapeDtypeStruct((N, TOP_K), jnp.int32),
          jax.ShapeDtypeStruct((N, TOP_K), jnp.float32),
      ),
  )(parts, isr_in, W_conv, b_conv, gate_W, gate_b)


A2 = N * TOP_K
BB = 256
PB = 36864
NB = PB // BB
GB = 128
GROWS = PB // NW
NPAD2 = 10240
CTOK = NPAD2 // NW
CB = 40


def _gather_rows_body(h_hbm, tok_hbm, out, tidx, rows, sem):
  c = lax.axis_index("c")
  s = lax.axis_index("s")
  w = s * NC + c
  base = w * GROWS

  def body(i, _):
    r0 = base + i * GB
    pltpu.sync_copy(tok_hbm.at[pl.ds(r0, GB)], tidx)
    pltpu.async_copy(h_hbm.at[tidx], rows, sem).wait()
    pltpu.sync_copy(rows, out.at[pl.ds(r0, GB)])
    return 0

  lax.fori_loop(0, GROWS // GB, body, 0)


def _gather_rows(h, tok_padded):
  k = pl.kernel(
      _gather_rows_body,
      out_type=jax.ShapeDtypeStruct((PB, D), jnp.float32),
      mesh=_mesh(),
      scratch_types=[
          pltpu.VMEM((GB,), jnp.int32),
          pltpu.VMEM((GB, D), jnp.float32),
          pltpu.SemaphoreType.DMA,
      ],
  )
  return k(h, tok_padded)


def _ffn_body(be_ref, x_ref, g_ref, w1_ref, b1_ref, w2_ref, b2_ref, y_ref):
  x = x_ref[...]
  a = jnp.dot(x, w1_ref[0], preferred_element_type=jnp.float32)
  a = a + b1_ref[0]
  a = 0.5 * a * (1.0 + lax.erf(a * 0.7071067811865476))
  y = jnp.dot(a, w2_ref[0], preferred_element_type=jnp.float32)
  y = y + b2_ref[0]
  y_ref[...] = y * g_ref[...]


def _ffn(x_sorted, gate_padded, block_expert, W1, b1, W2, b2):
  grid_spec = pltpu.PrefetchScalarGridSpec(
      num_scalar_prefetch=1,
      grid=(NB,),
      in_specs=[
          pl.BlockSpec((BB, D), lambda b, be: (b, 0)),
          pl.BlockSpec((BB, 1), lambda b, be: (b, 0)),
          pl.BlockSpec((1, D, H), lambda b, be: (be[b], 0, 0)),
          pl.BlockSpec((1, 1, H), lambda b, be: (be[b], 0, 0)),
          pl.BlockSpec((1, H, D), lambda b, be: (be[b], 0, 0)),
          pl.BlockSpec((1, 1, D), lambda b, be: (be[b], 0, 0)),
      ],
      out_specs=pl.BlockSpec((BB, D), lambda b, be: (b, 0)),
  )
  return pl.pallas_call(
      _ffn_body,
      grid_spec=grid_spec,
      out_shape=jax.ShapeDtypeStruct((PB, D), jnp.float32),
  )(block_expert, x_sorted, gate_padded.reshape(PB, 1),
    W1, b1.reshape(NUM_EXPERTS, 1, H), W2, b2.reshape(NUM_EXPERTS, 1, D))


def _combine_body(y_hbm, inv_hbm, out, iidx, rows, obuf, sem):
  c = lax.axis_index("c")
  s = lax.axis_index("s")
  w = s * NC + c
  tbase = w * CTOK

  def body(j, _):
    pltpu.sync_copy(inv_hbm.at[pl.ds(2 * tbase + j * 2 * CB, 2 * CB)], iidx)
    pltpu.async_copy(y_hbm.at[iidx], rows, sem).wait()

    def tok(i, _):
      def col(cc, _):
        a = rows[2 * i, pl.ds(cc * LANES, LANES)]
        b = rows[2 * i + 1, pl.ds(cc * LANES, LANES)]
        obuf[i, pl.ds(cc * LANES, LANES)] = a + b
        return 0
      lax.fori_loop(0, D // LANES, col, 0)
      return 0

    lax.fori_loop(0, CB, tok, 0)
    pltpu.sync_copy(obuf, out.at[pl.ds(tbase + j * CB, CB)])
    return 0

  lax.fori_loop(0, CTOK // CB, body, 0)


def _combine(y_sorted, invpos_padded):
  k = pl.kernel(
      _combine_body,
      out_type=jax.ShapeDtypeStruct((NPAD2, D), jnp.float32),
      mesh=_mesh(),
      scratch_types=[
          pltpu.VMEM((2 * CB,), jnp.int32),
          pltpu.VMEM((2 * CB, D), jnp.float32),
          pltpu.VMEM((CB, D), jnp.float32),
          pltpu.SemaphoreType.DMA,
      ],
  )
  return k(y_sorted, invpos_padded)


def _bn_body(h_ref, moe_ref, gamma_ref, beta_ref, out_ref):
  z = h_ref[...] + moe_ref[...]
  mean = jnp.mean(z, axis=0, keepdims=True)
  zc = z - mean
  var = jnp.mean(zc * zc, axis=0, keepdims=True)
  out_ref[...] = zc * lax.rsqrt(var + 1e-5) * gamma_ref[...][None, :] \
      + beta_ref[...][None, :]


def _batchnorm(h, moe, gamma, beta):
  return pl.pallas_call(
      _bn_body,
      out_shape=jax.ShapeDtypeStruct((N, D), jnp.float32),
  )(h, moe, gamma, beta)


def kernel(feats, edge_index, W_conv, b_conv, gate_W, gate_b, W1, b1, W2, b2,
           bn_gamma, bn_beta):
  src = edge_index[0]
  dst = edge_index[1]
  cnts = _degrees(src, dst)
  hpre, isr_in = _scale(cnts, feats)
  parts = _aggregate(hpre, src, dst)[:, :N, :]
  h, idx2, gates = _conv_gate(parts, isr_in, W_conv, b_conv, gate_W, gate_b)

  ex = idx2.reshape(A2)
  gflat = gates.reshape(A2)
  perm = jnp.argsort(ex, stable=True).astype(jnp.int32)
  ex_sorted = jnp.take(ex, perm)
  offsets = jnp.searchsorted(ex_sorted, jnp.arange(NUM_EXPERTS, dtype=jnp.int32),
                             side="left").astype(jnp.int32)
  counts = jnp.diff(jnp.append(offsets, A2))
  pcounts = ((counts + BB - 1) // BB) * BB
  poffsets = jnp.concatenate([jnp.zeros((1,), jnp.int32),
                              jnp.cumsum(pcounts)]).astype(jnp.int32)
  shift = poffsets[:NUM_EXPERTS] - offsets
  pos = jnp.arange(A2, dtype=jnp.int32) + jnp.take(shift, ex_sorted)
  tok_padded = (jnp.arange(PB, dtype=jnp.int32) % N).at[pos].set(perm // TOP_K)
  gate_padded = jnp.zeros((PB,), jnp.float32).at[pos].set(jnp.take(gflat, perm))
  invpos = jnp.zeros((A2,), jnp.int32).at[perm].set(pos)
  invpos_padded = jnp.concatenate(
      [invpos, jnp.arange(TOP_K * NPAD2 - A2, dtype=jnp.int32)])
  block_expert = jnp.clip(
      jnp.searchsorted(poffsets, jnp.arange(NB, dtype=jnp.int32) * BB,
                       side="right").astype(jnp.int32) - 1, 0, NUM_EXPERTS - 1)

  x_sorted = _gather_rows(h, tok_padded)
  y_sorted = _ffn(x_sorted, gate_padded, block_expert, W1, b1, W2, b2)
  moe = _combine(y_sorted, invpos_padded)[:N]
  return _batchnorm(h, moe, bn_gamma, bn_beta)
